# Initial kernel scaffold; baseline (speedup 1.0000x reference)
import functools

import jax
import jax.numpy as jnp
from jax import lax
from jax.experimental import pallas as pl
from jax.experimental.pallas import tpu as pltpu
from jax.experimental.pallas import tpu_sc as plsc

C = 32
HC = 16
LAYERS = 8
NGRAPH = 128
HIDDEN = 1024

NC = 2
NT = 16
CHUNK = 128
GW = 8

N = 100000
E = 1600000
RPT = N // NT
CPT = (-(-E // (NT * CHUNK)) + GW - 1) // GW * GW
NGROUP = CPT // GW
EPAD = NT * CPT * CHUNK
ACC_ROWS = N + 8

_mesh = plsc.VectorSubcoreMesh(core_axis_name="c", subcore_axis_name="s")


@functools.partial(
    pl.kernel,
    out_type=jax.ShapeDtypeStruct((N, C), jnp.float32),
    mesh=_mesh,
    scratch_types=[
        pltpu.VMEM((GW, CHUNK), jnp.int32),
        pltpu.VMEM((GW, CHUNK), jnp.int32),
        pltpu.VMEM((GW, CHUNK, HC), jnp.float32),
        pltpu.VMEM_SHARED((ACC_ROWS, HC), jnp.float32),
        pltpu.SemaphoreType.DMA,
        pltpu.SemaphoreType.DMA,
    ],
)
def _sc_aggregate(gidx, sidx, x16, x32, out, gbuf, sbuf, rows, acc, gsem, ssem):
    cid = lax.axis_index("c")
    tid = lax.axis_index("s")
    r0 = tid * RPT

    @pl.when(cid == 0)
    def _():
        pltpu.sync_copy(x32.at[pl.ds(r0, RPT), pl.ds(0, HC)],
                        acc.at[pl.ds(r0, RPT)])

    @pl.when(cid == 1)
    def _():
        pltpu.sync_copy(x32.at[pl.ds(r0, RPT), pl.ds(HC, HC)],
                        acc.at[pl.ds(r0, RPT)])

    plsc.subcore_barrier()

    @pl.loop(0, NGROUP)
    def _(g):
        roff = tid * CPT + g * GW
        pltpu.sync_copy(sidx.at[pl.ds(roff, GW)], sbuf)
        pltpu.sync_copy(gidx.at[pl.ds(cid * (NT * CPT) + roff, GW)], gbuf)
        gathers = []
        for j in range(GW):
            gathers.append(
                pltpu.async_copy(x16.at[gbuf.at[j]], rows.at[j], gsem))
        for d in gathers:
            d.wait()
        scatters = []
        for j in range(GW):
            scatters.append(
                pltpu.async_copy(rows.at[j], acc.at[sbuf.at[j]], ssem,
                                 add=True))
        for d in scatters:
            d.wait()

    plsc.subcore_barrier()

    @pl.when(cid == 0)
    def _():
        pltpu.sync_copy(acc.at[pl.ds(r0, RPT)],
                        out.at[pl.ds(r0, RPT), pl.ds(0, HC)])

    @pl.when(cid == 1)
    def _():
        pltpu.sync_copy(acc.at[pl.ds(r0, RPT)],
                        out.at[pl.ds(r0, RPT), pl.ds(HC, HC)])


@functools.partial(
    pl.kernel,
    out_type=jax.ShapeDtypeStruct((N, C), jnp.float32),
    mesh=_mesh,
    scratch_types=[
        pltpu.VMEM((GW, CHUNK), jnp.int32),
        pltpu.VMEM((CHUNK, HC), jnp.float32),
        pltpu.VMEM_SHARED((ACC_ROWS, HC), jnp.float32),
        pltpu.SemaphoreType.DMA,
    ],
)
def _sc_degrees(sidx2, ones_rows, zeros_half, out, sbuf, crows, acc, ssem):
    cid = lax.axis_index("c")
    tid = lax.axis_index("s")
    r0 = tid * RPT

    pltpu.sync_copy(ones_rows, crows)
    pltpu.sync_copy(zeros_half.at[pl.ds(r0, RPT)], acc.at[pl.ds(r0, RPT)])
    plsc.subcore_barrier()

    @pl.loop(0, NGROUP)
    def _(g):
        roff = cid * (NT * CPT) + tid * CPT + g * GW
        pltpu.sync_copy(sidx2.at[pl.ds(roff, GW)], sbuf)
        scatters = []
        for j in range(GW):
            scatters.append(
                pltpu.async_copy(crows, acc.at[sbuf.at[j]], ssem, add=True))
        for d in scatters:
            d.wait()

    plsc.subcore_barrier()

    @pl.when(cid == 0)
    def _():
        pltpu.sync_copy(acc.at[pl.ds(r0, RPT)],
                        out.at[pl.ds(r0, RPT), pl.ds(0, HC)])

    @pl.when(cid == 1)
    def _():
        pltpu.sync_copy(acc.at[pl.ds(r0, RPT)],
                        out.at[pl.ds(r0, RPT), pl.ds(HC, HC)])


RB = 2000
NB = N // RB


def _init_body(nodes_ref, emb_ref, deg_ref, x0_ref, nf_ref, nb_ref):
    nid = nodes_ref[...]
    emb = emb_ref[...]
    acc = jnp.zeros((RB, C), jnp.float32)
    for t in range(7):
        acc += (nid == t).astype(jnp.float32) * emb[t:t + 1, :]
    x0_ref[...] = acc
    dd = deg_ref[...]
    nf_ref[...] = 1.0 / (1.0 + dd[:, 0:1])
    nb_ref[...] = 1.0 / (1.0 + dd[:, HC:HC + 1])


def _tc_init(nodes2, emb8, degdump):
    return pl.pallas_call(
        _init_body,
        grid=(NB,),
        in_specs=[
            pl.BlockSpec((RB, 1), lambda i: (i, 0)),
            pl.BlockSpec((8, C), lambda i: (0, 0)),
            pl.BlockSpec((RB, C), lambda i: (i, 0)),
        ],
        out_specs=[
            pl.BlockSpec((RB, C), lambda i: (i, 0)),
            pl.BlockSpec((RB, 1), lambda i: (i, 0)),
            pl.BlockSpec((RB, 1), lambda i: (i, 0)),
        ],
        out_shape=[
            jax.ShapeDtypeStruct((N, C), jnp.float32),
            jax.ShapeDtypeStruct((N, 1), jnp.float32),
            jax.ShapeDtypeStruct((N, 1), jnp.float32),
        ],
    )(nodes2, emb8, degdump)


def _dense_body(x_ref, af_ref, ab_ref, nf_ref, nb_ref, wf_ref, wb_ref, o_ref):
    fwd = jnp.dot(nf_ref[...] * af_ref[...], wf_ref[...],
                  preferred_element_type=jnp.float32)
    bwd = jnp.dot(nb_ref[...] * ab_ref[...], wb_ref[...],
                  preferred_element_type=jnp.float32)
    o_ref[...] = x_ref[...] + jnp.maximum(fwd, 0.0) + jnp.maximum(bwd, 0.0)


def _tc_dense(x, aggf, aggb, nf, nb, wf, wb):
    return pl.pallas_call(
        _dense_body,
        grid=(NB,),
        in_specs=[
            pl.BlockSpec((RB, C), lambda i: (i, 0)),
            pl.BlockSpec((RB, C), lambda i: (i, 0)),
            pl.BlockSpec((RB, C), lambda i: (i, 0)),
            pl.BlockSpec((RB, 1), lambda i: (i, 0)),
            pl.BlockSpec((RB, 1), lambda i: (i, 0)),
            pl.BlockSpec((C, C), lambda i: (0, 0)),
            pl.BlockSpec((C, C), lambda i: (0, 0)),
        ],
        out_specs=pl.BlockSpec((RB, C), lambda i: (i, 0)),
        out_shape=jax.ShapeDtypeStruct((N, C), jnp.float32),
    )(x, aggf, aggb, nf, nb, wf, wb)


def _readout_body(x_ref, b_ref, hw_ref, hb_ref, ow_ref, o_ref, pooled, counts):
    i = pl.program_id(0)

    @pl.when(i == 0)
    def _():
        pooled[...] = jnp.zeros((NGRAPH, C), jnp.float32)
        counts[...] = jnp.zeros((NGRAPH, 1), jnp.float32)

    gids = b_ref[...]
    onehot = (gids == lax.broadcasted_iota(jnp.int32, (1, NGRAPH), 1)
              ).astype(jnp.float32)
    pooled[...] += lax.dot_general(
        onehot, x_ref[...], (((0,), (0,)), ((), ())),
        preferred_element_type=jnp.float32)
    counts[...] += lax.dot_general(
        onehot, jnp.ones((RB, 1), jnp.float32), (((0,), (0,)), ((), ())),
        preferred_element_type=jnp.float32)

    @pl.when(i == NB - 1)
    def _():
        mean = pooled[...] * (1.0 / counts[...])
        h = jnp.maximum(
            jnp.dot(mean, hw_ref[...], preferred_element_type=jnp.float32)
            + hb_ref[...], 0.0)
        o_ref[...] = lax.dot_general(
            h, ow_ref[...], (((1,), (1,)), ((), ())),
            preferred_element_type=jnp.float32)


def _tc_readout(x, batch2, hidden_w, hidden_b2, output_w2):
    return pl.pallas_call(
        _readout_body,
        grid=(NB,),
        in_specs=[
            pl.BlockSpec((RB, C), lambda i: (i, 0)),
            pl.BlockSpec((RB, 1), lambda i: (i, 0)),
            pl.BlockSpec((C, HIDDEN), lambda i: (0, 0)),
            pl.BlockSpec((1, HIDDEN), lambda i: (0, 0)),
            pl.BlockSpec((1, HIDDEN), lambda i: (0, 0)),
        ],
        out_specs=pl.BlockSpec((NGRAPH, 1), lambda i: (0, 0)),
        out_shape=jax.ShapeDtypeStruct((NGRAPH, 1), jnp.float32),
        scratch_shapes=[
            pltpu.VMEM((NGRAPH, C), jnp.float32),
            pltpu.VMEM((NGRAPH, 1), jnp.float32),
        ],
    )(x, batch2, hidden_w, hidden_b2, output_w2)


def kernel(nodes, sources, targets, batch, embedding, W_out, W_back,
           hidden_w, hidden_b, output_w):
    pad = EPAD - E
    src_g = jnp.concatenate([sources, jnp.zeros((pad,), jnp.int32)])
    tgt_g = jnp.concatenate([targets, jnp.zeros((pad,), jnp.int32)])
    src_s = jnp.concatenate([sources, jnp.full((pad,), N, jnp.int32)])
    tgt_s = jnp.concatenate([targets, jnp.full((pad,), N, jnp.int32)])

    gidx_f = jnp.concatenate([2 * src_g, 2 * src_g + 1]).reshape(-1, CHUNK)
    gidx_b = jnp.concatenate([2 * tgt_g, 2 * tgt_g + 1]).reshape(-1, CHUNK)
    sidx_f = tgt_s.reshape(-1, CHUNK)
    sidx_b = src_s.reshape(-1, CHUNK)
    sidx_deg = jnp.concatenate([tgt_s, src_s]).reshape(-1, CHUNK)

    ones_rows = jnp.ones((CHUNK, HC), jnp.float32)
    zeros_half = jnp.zeros((N, HC), jnp.float32)
    degdump = _sc_degrees(sidx_deg, ones_rows, zeros_half)

    emb8 = jnp.concatenate([embedding,
                            jnp.zeros((1, C), jnp.float32)], axis=0)
    x, nf, nb = _tc_init(nodes.reshape(N, 1), emb8, degdump)

    for i in range(LAYERS):
        x16 = x.reshape(2 * N, HC)
        aggf = _sc_aggregate(gidx_f, sidx_f, x16, x)
        aggb = _sc_aggregate(gidx_b, sidx_b, x16, x)
        x = _tc_dense(x, aggf, aggb, nf, nb, W_out[i], W_back[i])

    out = _tc_readout(x, batch.reshape(N, 1), hidden_w,
                      hidden_b.reshape(1, HIDDEN),
                      output_w.reshape(1, HIDDEN))
    return out.reshape(NGRAPH)


# speedup vs baseline: 11.0218x; 11.0218x over previous
import functools

import jax
import jax.numpy as jnp
from jax import lax
from jax.experimental import pallas as pl
from jax.experimental.pallas import tpu as pltpu
from jax.experimental.pallas import tpu_sc as plsc

C = 32
HC = 16
LAYERS = 8
NGRAPH = 128
HIDDEN = 1024

NC = 2
NT = 16
CHUNK = 128
GW = 8

N = 100000
E = 1600000
RPT_A = 6256
RPT_LAST = N - (NT - 1) * RPT_A
CPT = (-(-E // (NT * CHUNK)) + GW - 1) // GW * GW
NGROUP = CPT // GW
EPAD = NT * CPT * CHUNK
ACC_ROWS = N + 8

_mesh = plsc.VectorSubcoreMesh(core_axis_name="c", subcore_axis_name="s")
_sc_params = pltpu.CompilerParams(use_tc_tiling_on_sc=False)


def _per_tile_rows(tid, fn):
    @pl.when(tid < NT - 1)
    def _():
        fn(tid * RPT_A, RPT_A)

    @pl.when(tid == NT - 1)
    def _():
        fn((NT - 1) * RPT_A, RPT_LAST)


@functools.partial(
    pl.kernel,
    out_type=jax.ShapeDtypeStruct((NC, N, HC), jnp.float32),
    mesh=_mesh,
    scratch_types=[
        pltpu.VMEM((GW, CHUNK), jnp.int32),
        pltpu.VMEM((GW, CHUNK), jnp.int32),
        pltpu.VMEM((GW, CHUNK, HC), jnp.float32),
        pltpu.VMEM_SHARED((ACC_ROWS, HC), jnp.float32),
        pltpu.SemaphoreType.DMA,
        pltpu.SemaphoreType.DMA,
    ],
    compiler_params=_sc_params,
)
def _sc_aggregate(gidx, sidx, x16, zeros_half, out3, gbuf, sbuf, rows, acc,
                  gsem, ssem):
    cid = lax.axis_index("c")
    tid = lax.axis_index("s")

    _per_tile_rows(tid, lambda r0, nr: pltpu.sync_copy(
        zeros_half.at[pl.ds(r0, nr)], acc.at[pl.ds(r0, nr)]))
    plsc.subcore_barrier()

    @pl.loop(0, NGROUP)
    def _(g):
        roff = tid * CPT + g * GW
        pltpu.sync_copy(sidx.at[pl.ds(roff, GW)], sbuf)
        pltpu.sync_copy(gidx.at[pl.ds(cid * (NT * CPT) + roff, GW)], gbuf)
        gathers = []
        for j in range(GW):
            gathers.append(
                pltpu.async_copy(x16.at[gbuf.at[j]], rows.at[j], gsem))
        for d in gathers:
            d.wait()
        scatters = []
        for j in range(GW):
            scatters.append(
                pltpu.async_copy(rows.at[j], acc.at[sbuf.at[j]], ssem,
                                 add=True))
        for d in scatters:
            d.wait()

    plsc.subcore_barrier()
    _per_tile_rows(tid, lambda r0, nr: pltpu.sync_copy(
        acc.at[pl.ds(r0, nr)], out3.at[cid, pl.ds(r0, nr)]))


@functools.partial(
    pl.kernel,
    out_type=jax.ShapeDtypeStruct((NC, N, HC), jnp.float32),
    mesh=_mesh,
    scratch_types=[
        pltpu.VMEM((GW, CHUNK), jnp.int32),
        pltpu.VMEM((CHUNK, HC), jnp.float32),
        pltpu.VMEM_SHARED((ACC_ROWS, HC), jnp.float32),
        pltpu.SemaphoreType.DMA,
    ],
    compiler_params=_sc_params,
)
def _sc_degrees(sidx2, ones_rows, zeros_half, out3, sbuf, crows, acc, ssem):
    cid = lax.axis_index("c")
    tid = lax.axis_index("s")

    pltpu.sync_copy(ones_rows, crows)
    _per_tile_rows(tid, lambda r0, nr: pltpu.sync_copy(
        zeros_half.at[pl.ds(r0, nr)], acc.at[pl.ds(r0, nr)]))
    plsc.subcore_barrier()

    @pl.loop(0, NGROUP)
    def _(g):
        roff = cid * (NT * CPT) + tid * CPT + g * GW
        pltpu.sync_copy(sidx2.at[pl.ds(roff, GW)], sbuf)
        scatters = []
        for j in range(GW):
            scatters.append(
                pltpu.async_copy(crows, acc.at[sbuf.at[j]], ssem, add=True))
        for d in scatters:
            d.wait()

    plsc.subcore_barrier()
    _per_tile_rows(tid, lambda r0, nr: pltpu.sync_copy(
        acc.at[pl.ds(r0, nr)], out3.at[cid, pl.ds(r0, nr)]))


RB = 2000
NB = N // RB


def _init_body(nodes_ref, din_ref, dout_ref, emb_ref, x0_ref, nf_ref, nb_ref):
    nid = nodes_ref[...]
    emb = emb_ref[...]
    acc = jnp.zeros((RB, C), jnp.float32)
    for t in range(7):
        acc += (nid == t).astype(jnp.float32) * emb[t:t + 1, :]
    x0_ref[...] = acc
    nf_ref[...] = 1.0 / (1.0 + din_ref[0][:, 0:1])
    nb_ref[...] = 1.0 / (1.0 + dout_ref[0][:, 0:1])


def _tc_init(nodes2, deg3, emb8):
    return pl.pallas_call(
        _init_body,
        grid=(NB,),
        in_specs=[
            pl.BlockSpec((RB, 1), lambda i: (i, 0)),
            pl.BlockSpec((1, RB, HC), lambda i: (0, i, 0)),
            pl.BlockSpec((1, RB, HC), lambda i: (1, i, 0)),
            pl.BlockSpec((8, C), lambda i: (0, 0)),
        ],
        out_specs=[
            pl.BlockSpec((RB, C), lambda i: (i, 0)),
            pl.BlockSpec((RB, 1), lambda i: (i, 0)),
            pl.BlockSpec((RB, 1), lambda i: (i, 0)),
        ],
        out_shape=[
            jax.ShapeDtypeStruct((N, C), jnp.float32),
            jax.ShapeDtypeStruct((N, 1), jnp.float32),
            jax.ShapeDtypeStruct((N, 1), jnp.float32),
        ],
    )(nodes2, deg3, deg3, emb8)


def _dense_body(x_ref, aflo_ref, afhi_ref, ablo_ref, abhi_ref,
                nf_ref, nb_ref, wf_ref, wb_ref, o_ref):
    xb = x_ref[...]
    af = jnp.concatenate([aflo_ref[0], afhi_ref[0]], axis=1)
    ab = jnp.concatenate([ablo_ref[0], abhi_ref[0]], axis=1)
    fwd = jnp.dot(nf_ref[...] * (xb + af), wf_ref[...],
                  preferred_element_type=jnp.float32)
    bwd = jnp.dot(nb_ref[...] * (xb + ab), wb_ref[...],
                  preferred_element_type=jnp.float32)
    o_ref[...] = xb + jnp.maximum(fwd, 0.0) + jnp.maximum(bwd, 0.0)


def _tc_dense(x, aggf3, aggb3, nf, nb, wf, wb):
    return pl.pallas_call(
        _dense_body,
        grid=(NB,),
        in_specs=[
            pl.BlockSpec((RB, C), lambda i: (i, 0)),
            pl.BlockSpec((1, RB, HC), lambda i: (0, i, 0)),
            pl.BlockSpec((1, RB, HC), lambda i: (1, i, 0)),
            pl.BlockSpec((1, RB, HC), lambda i: (0, i, 0)),
            pl.BlockSpec((1, RB, HC), lambda i: (1, i, 0)),
            pl.BlockSpec((RB, 1), lambda i: (i, 0)),
            pl.BlockSpec((RB, 1), lambda i: (i, 0)),
            pl.BlockSpec((C, C), lambda i: (0, 0)),
            pl.BlockSpec((C, C), lambda i: (0, 0)),
        ],
        out_specs=pl.BlockSpec((RB, C), lambda i: (i, 0)),
        out_shape=jax.ShapeDtypeStruct((N, C), jnp.float32),
    )(x, aggf3, aggf3, aggb3, aggb3, nf, nb, wf, wb)


def _readout_body(x_ref, b_ref, hw_ref, hb_ref, ow_ref, o_ref, pooled, counts):
    i = pl.program_id(0)

    @pl.when(i == 0)
    def _():
        pooled[...] = jnp.zeros((NGRAPH, C), jnp.float32)
        counts[...] = jnp.zeros((NGRAPH, 1), jnp.float32)

    gids = b_ref[...]
    onehot = (gids == lax.broadcasted_iota(jnp.int32, (1, NGRAPH), 1)
              ).astype(jnp.float32)
    pooled[...] += lax.dot_general(
        onehot, x_ref[...], (((0,), (0,)), ((), ())),
        preferred_element_type=jnp.float32)
    counts[...] += lax.dot_general(
        onehot, jnp.ones((RB, 1), jnp.float32), (((0,), (0,)), ((), ())),
        preferred_element_type=jnp.float32)

    @pl.when(i == NB - 1)
    def _():
        mean = pooled[...] * (1.0 / counts[...])
        h = jnp.maximum(
            jnp.dot(mean, hw_ref[...], preferred_element_type=jnp.float32)
            + hb_ref[...], 0.0)
        o_ref[...] = lax.dot_general(
            h, ow_ref[...], (((1,), (1,)), ((), ())),
            preferred_element_type=jnp.float32)


def _tc_readout(x, batch2, hidden_w, hidden_b2, output_w2):
    return pl.pallas_call(
        _readout_body,
        grid=(NB,),
        in_specs=[
            pl.BlockSpec((RB, C), lambda i: (i, 0)),
            pl.BlockSpec((RB, 1), lambda i: (i, 0)),
            pl.BlockSpec((C, HIDDEN), lambda i: (0, 0)),
            pl.BlockSpec((1, HIDDEN), lambda i: (0, 0)),
            pl.BlockSpec((1, HIDDEN), lambda i: (0, 0)),
        ],
        out_specs=pl.BlockSpec((NGRAPH, 1), lambda i: (0, 0)),
        out_shape=jax.ShapeDtypeStruct((NGRAPH, 1), jnp.float32),
        scratch_shapes=[
            pltpu.VMEM((NGRAPH, C), jnp.float32),
            pltpu.VMEM((NGRAPH, 1), jnp.float32),
        ],
    )(x, batch2, hidden_w, hidden_b2, output_w2)


def kernel(nodes, sources, targets, batch, embedding, W_out, W_back,
           hidden_w, hidden_b, output_w):
    pad = EPAD - E
    src_g = jnp.concatenate([sources, jnp.zeros((pad,), jnp.int32)])
    tgt_g = jnp.concatenate([targets, jnp.zeros((pad,), jnp.int32)])
    src_s = jnp.concatenate([sources, jnp.full((pad,), N, jnp.int32)])
    tgt_s = jnp.concatenate([targets, jnp.full((pad,), N, jnp.int32)])

    gidx_f = jnp.concatenate([2 * src_g, 2 * src_g + 1]).reshape(-1, CHUNK)
    gidx_b = jnp.concatenate([2 * tgt_g, 2 * tgt_g + 1]).reshape(-1, CHUNK)
    sidx_f = tgt_s.reshape(-1, CHUNK)
    sidx_b = src_s.reshape(-1, CHUNK)
    sidx_deg = jnp.concatenate([tgt_s, src_s]).reshape(-1, CHUNK)

    ones_rows = jnp.ones((CHUNK, HC), jnp.float32)
    zeros_half = jnp.zeros((N, HC), jnp.float32)
    deg3 = _sc_degrees(sidx_deg, ones_rows, zeros_half)

    emb8 = jnp.concatenate([embedding,
                            jnp.zeros((1, C), jnp.float32)], axis=0)
    x, nf, nb = _tc_init(nodes.reshape(N, 1), deg3, emb8)

    for i in range(LAYERS):
        x16 = x.reshape(2 * N, HC)
        aggf3 = _sc_aggregate(gidx_f, sidx_f, x16, zeros_half)
        aggb3 = _sc_aggregate(gidx_b, sidx_b, x16, zeros_half)
        x = _tc_dense(x, aggf3, aggb3, nf, nb, W_out[i], W_back[i])

    out = _tc_readout(x, batch.reshape(N, 1), hidden_w,
                      hidden_b.reshape(1, HIDDEN),
                      output_w.reshape(1, HIDDEN))
    return out.reshape(NGRAPH)
